# baseline (speedup 1.0000x reference)
import functools

import jax
import jax.numpy as jnp
from jax import lax
from jax.experimental import pallas as pl
from jax.experimental.pallas import tpu as pltpu
from jax.experimental.pallas import tpu_sc as plsc

_D = 512
_B = 4096
_L = 77
_LP = 80
_NC = 2
_NS = 16
_NW = _NC * _NS
_BPW = _B // _NW
_CH = _D // 16


def _sums_sc(ids, maskx, table):
    mesh = plsc.VectorSubcoreMesh(core_axis_name="c", subcore_axis_name="s")

    @functools.partial(
        pl.kernel,
        mesh=mesh,
        out_type=jax.ShapeDtypeStruct((_B, _D), jnp.float32),
        scratch_types=[
            pltpu.VMEM((_LP,), jnp.int32),
            pltpu.VMEM((_LP, 16), jnp.float32),
            pltpu.VMEM((_LP, _D), jnp.float32),
            pltpu.VMEM((_D,), jnp.float32),
            pltpu.SemaphoreType.DMA,
        ],
    )
    def k(ids_hbm, maskx_hbm, table_hbm, out_hbm, idx_v, wexp_v, rows_v,
          acc_v, sem):
        wid = lax.axis_index("s") * _NC + lax.axis_index("c")
        base = wid * _BPW

        def body(i, carry):
            row = base + i
            pltpu.sync_copy(ids_hbm.at[row], idx_v)
            pltpu.sync_copy(maskx_hbm.at[row], wexp_v)
            pltpu.async_copy(table_hbm.at[idx_v], rows_v, sem).wait()

            zero = jnp.zeros((16,), jnp.float32)
            for c in range(_CH):
                acc_v[pl.ds(c * 16, 16)] = zero

            def lbody(l, carry2):
                w = wexp_v[l, :]
                for c in range(_CH):
                    plsc.addupdate(acc_v.at[pl.ds(c * 16, 16)],
                                   rows_v[l, pl.ds(c * 16, 16)] * w)
                return carry2

            lax.fori_loop(0, _LP, lbody, jnp.int32(0))
            pltpu.sync_copy(acc_v, out_hbm.at[row])
            return carry

        lax.fori_loop(0, _BPW, body, jnp.int32(0))

    return k(ids, maskx, table)


def _mm_body(s_ref, m_ref, w_ref, b_ref, o_ref):
    cnt = jnp.sum(m_ref[...].astype(jnp.float32), axis=1, keepdims=True)
    pooled = s_ref[...] / jnp.maximum(cnt, jnp.float32(1e-9))
    o_ref[...] = (
        lax.dot_general(pooled, w_ref[...],
                        (((1,), (1,)), ((), ())),
                        preferred_element_type=jnp.float32)
        + b_ref[0:1, :]
    )


def _project_tc(sums, mask_p, W, b):
    tb = 512
    b2 = jnp.tile(b[None, :], (8, 1))
    return pl.pallas_call(
        _mm_body,
        grid=(_B // tb,),
        in_specs=[
            pl.BlockSpec((tb, _D), lambda i: (i, 0)),
            pl.BlockSpec((tb, _LP), lambda i: (i, 0)),
            pl.BlockSpec((_D, _D), lambda i: (0, 0)),
            pl.BlockSpec((8, _D), lambda i: (0, 0)),
        ],
        out_specs=pl.BlockSpec((tb, _D), lambda i: (i, 0)),
        out_shape=jax.ShapeDtypeStruct((_B, _D), jnp.float32),
    )(sums, mask_p, W, b2)


def kernel(input_ids, attention_mask, emb_table, W, b):
    ids_p = jnp.pad(input_ids, ((0, 0), (0, _LP - _L)))
    mask_p = jnp.pad(attention_mask, ((0, 0), (0, _LP - _L)))
    maskx = jnp.broadcast_to(
        mask_p.astype(jnp.float32)[:, :, None], (_B, _LP, 16))
    sums = _sums_sc(ids_p, maskx, emb_table)
    out = _project_tc(sums, mask_p, W, b)
    return out[:, None, :]


# speedup vs baseline: 1.1580x; 1.1580x over previous
import functools

import jax
import jax.numpy as jnp
from jax import lax
from jax.experimental import pallas as pl
from jax.experimental.pallas import tpu as pltpu
from jax.experimental.pallas import tpu_sc as plsc

_D = 512
_B = 4096
_L = 77
_LP = 80
_NC = 2
_NS = 16
_NW = _NC * _NS
_BPW = _B // _NW
_CH = _D // 16


def _sums_sc(ids, maskx, table):
    mesh = plsc.VectorSubcoreMesh(core_axis_name="c", subcore_axis_name="s")

    @functools.partial(
        pl.kernel,
        mesh=mesh,
        out_type=jax.ShapeDtypeStruct((_B, _D), jnp.float32),
        scratch_types=[
            pltpu.VMEM((_LP,), jnp.int32),
            pltpu.VMEM((_LP,), jnp.int32),
            pltpu.VMEM((_LP, 16), jnp.float32),
            pltpu.VMEM((_LP, 16), jnp.float32),
            pltpu.VMEM((_LP, _D), jnp.float32),
            pltpu.VMEM((_LP, _D), jnp.float32),
            pltpu.VMEM((_D,), jnp.float32),
            pltpu.SemaphoreType.DMA,
            pltpu.SemaphoreType.DMA,
        ],
    )
    def k(ids_hbm, maskx_hbm, table_hbm, out_hbm, idx_a, idx_b, wexp_a,
          wexp_b, rows_a, rows_b, acc_v, sem_a, sem_b):
        wid = lax.axis_index("s") * _NC + lax.axis_index("c")
        base = wid * _BPW

        def issue(row, idx_v, wexp_v, rows_v, sem):
            pltpu.sync_copy(ids_hbm.at[row], idx_v)
            pltpu.sync_copy(maskx_hbm.at[row], wexp_v)
            return pltpu.async_copy(table_hbm.at[idx_v], rows_v, sem)

        def consume(row, idx_v, wexp_v, rows_v, sem):
            pltpu.make_async_copy(table_hbm.at[idx_v], rows_v, sem).wait()
            zero = jnp.zeros((16,), jnp.float32)
            for c in range(_CH):
                acc_v[pl.ds(c * 16, 16)] = zero

            def lbody(l4, carry2):
                for u in range(4):
                    l = l4 * 4 + u
                    w = wexp_v[l, :]
                    for c in range(_CH):
                        plsc.addupdate(acc_v.at[pl.ds(c * 16, 16)],
                                       rows_v[l, pl.ds(c * 16, 16)] * w)
                return carry2

            lax.fori_loop(0, _LP // 4, lbody, jnp.int32(0))
            pltpu.sync_copy(acc_v, out_hbm.at[row])

        issue(base, idx_a, wexp_a, rows_a, sem_a)

        def body(i2, carry):
            row_a = base + 2 * i2
            row_b = row_a + 1
            issue(row_b, idx_b, wexp_b, rows_b, sem_b)
            consume(row_a, idx_a, wexp_a, rows_a, sem_a)
            row_n = jnp.minimum(row_a + 2, jnp.int32(_B - 1))
            issue(row_n, idx_a, wexp_a, rows_a, sem_a)
            consume(row_b, idx_b, wexp_b, rows_b, sem_b)
            return carry

        lax.fori_loop(0, _BPW // 2, body, jnp.int32(0))
        pltpu.make_async_copy(table_hbm.at[idx_a], rows_a, sem_a).wait()

    return k(ids, maskx, table)


def _mm_body(s_ref, m_ref, w_ref, b_ref, o_ref):
    cnt = jnp.sum(m_ref[...].astype(jnp.float32), axis=1, keepdims=True)
    pooled = s_ref[...] / jnp.maximum(cnt, jnp.float32(1e-9))
    o_ref[...] = (
        lax.dot_general(pooled, w_ref[...],
                        (((1,), (1,)), ((), ())),
                        preferred_element_type=jnp.float32)
        + b_ref[0:1, :]
    )


def _project_tc(sums, mask_p, W, b):
    tb = 512
    b2 = jnp.tile(b[None, :], (8, 1))
    return pl.pallas_call(
        _mm_body,
        grid=(_B // tb,),
        in_specs=[
            pl.BlockSpec((tb, _D), lambda i: (i, 0)),
            pl.BlockSpec((tb, _LP), lambda i: (i, 0)),
            pl.BlockSpec((_D, _D), lambda i: (0, 0)),
            pl.BlockSpec((8, _D), lambda i: (0, 0)),
        ],
        out_specs=pl.BlockSpec((tb, _D), lambda i: (i, 0)),
        out_shape=jax.ShapeDtypeStruct((_B, _D), jnp.float32),
    )(sums, mask_p, W, b2)


def kernel(input_ids, attention_mask, emb_table, W, b):
    ids_p = jnp.pad(input_ids, ((0, 0), (0, _LP - _L)))
    mask_p = jnp.pad(attention_mask, ((0, 0), (0, _LP - _L)))
    maskx = jnp.broadcast_to(
        mask_p.astype(jnp.float32)[:, :, None], (_B, _LP, 16))
    sums = _sums_sc(ids_p, maskx, emb_table)
    out = _project_tc(sums, mask_p, W, b)
    return out[:, None, :]


# speedup vs baseline: 2.3412x; 2.0218x over previous
import functools

import jax
import jax.numpy as jnp
from jax import lax
from jax.experimental import pallas as pl
from jax.experimental.pallas import tpu as pltpu
from jax.experimental.pallas import tpu_sc as plsc

_D = 512
_B = 4096
_L = 77
_LP = 80
_NC = 2
_NS = 16
_NW = _NC * _NS
_BPW = _B // _NW
_CH = _D // 16


def _sums_sc(ids, maskx, table):
    mesh = plsc.VectorSubcoreMesh(core_axis_name="c", subcore_axis_name="s")

    @functools.partial(
        pl.kernel,
        mesh=mesh,
        out_type=jax.ShapeDtypeStruct((_B, _D), jnp.float32),
        scratch_types=[
            pltpu.VMEM((_LP,), jnp.int32),
            pltpu.VMEM((_LP,), jnp.int32),
            pltpu.VMEM((_LP, 16), jnp.float32),
            pltpu.VMEM((_LP, 16), jnp.float32),
            pltpu.VMEM((_LP, _D), jnp.float32),
            pltpu.VMEM((_LP, _D), jnp.float32),
            pltpu.VMEM((_D,), jnp.float32),
            pltpu.SemaphoreType.DMA,
            pltpu.SemaphoreType.DMA,
        ],
    )
    def k(ids_hbm, maskx_hbm, table_hbm, out_hbm, idx_a, idx_b, wexp_a,
          wexp_b, rows_a, rows_b, acc_v, sem_a, sem_b):
        wid = lax.axis_index("s") * _NC + lax.axis_index("c")
        base = wid * _BPW

        def issue(row, idx_v, wexp_v, rows_v, sem):
            pltpu.sync_copy(ids_hbm.at[row], idx_v)
            pltpu.sync_copy(maskx_hbm.at[row], wexp_v)
            return pltpu.async_copy(table_hbm.at[idx_v], rows_v, sem)

        def consume(row, idx_v, wexp_v, rows_v, sem):
            pltpu.make_async_copy(table_hbm.at[idx_v], rows_v, sem).wait()

            def lbody(l, accs):
                w = wexp_v[l, :]
                return tuple(
                    accs[c] + rows_v[l, pl.ds(c * 16, 16)] * w
                    for c in range(_CH))

            init = tuple(jnp.zeros((16,), jnp.float32) for _ in range(_CH))
            accs = lax.fori_loop(0, _LP, lbody, init)
            for c in range(_CH):
                acc_v[pl.ds(c * 16, 16)] = accs[c]
            pltpu.sync_copy(acc_v, out_hbm.at[row])

        issue(base, idx_a, wexp_a, rows_a, sem_a)

        def body(i2, carry):
            row_a = base + 2 * i2
            row_b = row_a + 1
            issue(row_b, idx_b, wexp_b, rows_b, sem_b)
            consume(row_a, idx_a, wexp_a, rows_a, sem_a)
            row_n = jnp.minimum(row_a + 2, jnp.int32(_B - 1))
            issue(row_n, idx_a, wexp_a, rows_a, sem_a)
            consume(row_b, idx_b, wexp_b, rows_b, sem_b)
            return carry

        lax.fori_loop(0, _BPW // 2, body, jnp.int32(0))
        pltpu.make_async_copy(table_hbm.at[idx_a], rows_a, sem_a).wait()

    return k(ids, maskx, table)


def _mm_body(s_ref, m_ref, w_ref, b_ref, o_ref):
    cnt = jnp.sum(m_ref[...].astype(jnp.float32), axis=1, keepdims=True)
    pooled = s_ref[...] / jnp.maximum(cnt, jnp.float32(1e-9))
    o_ref[...] = (
        lax.dot_general(pooled, w_ref[...],
                        (((1,), (1,)), ((), ())),
                        preferred_element_type=jnp.float32)
        + b_ref[0:1, :]
    )


def _project_tc(sums, mask_p, W, b):
    tb = 512
    b2 = jnp.tile(b[None, :], (8, 1))
    return pl.pallas_call(
        _mm_body,
        grid=(_B // tb,),
        in_specs=[
            pl.BlockSpec((tb, _D), lambda i: (i, 0)),
            pl.BlockSpec((tb, _LP), lambda i: (i, 0)),
            pl.BlockSpec((_D, _D), lambda i: (0, 0)),
            pl.BlockSpec((8, _D), lambda i: (0, 0)),
        ],
        out_specs=pl.BlockSpec((tb, _D), lambda i: (i, 0)),
        out_shape=jax.ShapeDtypeStruct((_B, _D), jnp.float32),
    )(sums, mask_p, W, b2)


def kernel(input_ids, attention_mask, emb_table, W, b):
    ids_p = jnp.pad(input_ids, ((0, 0), (0, _LP - _L)))
    mask_p = jnp.pad(attention_mask, ((0, 0), (0, _LP - _L)))
    maskx = jnp.broadcast_to(
        mask_p.astype(jnp.float32)[:, :, None], (_B, _LP, 16))
    sums = _sums_sc(ids_p, maskx, emb_table)
    out = _project_tc(sums, mask_p, W, b)
    return out[:, None, :]
